# baseline (speedup 1.0000x reference)
import functools

import jax
import jax.numpy as jnp
from jax import lax
from jax.experimental import pallas as pl
from jax.experimental.pallas import tpu as pltpu
from jax.experimental.pallas import tpu_sc as plsc

N = 10000
D = 128
E = 320000
NC = 2
NS = 16
NW = NC * NS
L = 16
DH = 64
CHUNK = 128
EPC = 80
RPC = 40
EPC_IDX = EPC + 2
EPT = EPC * CHUNK
EPAD = NW * EPC_IDX * CHUNK
NPAD = 10112
RPT = NPAD // NS


def _sc_mesh():
    return plsc.VectorSubcoreMesh(core_axis_name="c", subcore_axis_name="s")


def _deg_body(dst_hbm, out_hbm, di_v, ones_v, stg_v, deg_sh, sem, ss0, ss1):
    c = lax.axis_index("c")
    s = lax.axis_index("s")
    sem_s = [ss0, ss1]
    pltpu.async_copy(dst_hbm.at[s, pl.ds(2 * c, 2)], di_v, sem)

    e0 = jnp.where(lax.iota(jnp.int32, L) == 0, 1.0, 0.0).astype(jnp.float32)

    def _fill(i, carry):
        ones_v[i, :] = e0
        return carry

    lax.fori_loop(0, CHUNK, _fill, 0, unroll=8)

    def _zero(i, carry):
        stg_v[i, :] = jnp.zeros((L,), jnp.float32)
        return carry

    lax.fori_loop(0, RPT, _zero, 0, unroll=8)
    pltpu.sync_copy(stg_v, deg_sh.at[pl.ds(s * RPT, RPT)])
    pltpu.make_async_copy(dst_hbm.at[s, pl.ds(2 * c, 2)], di_v, sem).wait()
    plsc.subcore_barrier()

    for b in (0, 1):
        pltpu.async_copy(ones_v, deg_sh.at[di_v.at[0, b]], sem_s[b], add=True)

    def _hist(g, carry):
        for b in (0, 1):
            j = 2 * g + b
            pltpu.make_async_copy(ones_v, deg_sh.at[di_v.at[(j - 2) // RPC, (j - 2) % RPC]],
                                  sem_s[b]).wait()
            pltpu.async_copy(ones_v, deg_sh.at[di_v.at[j // RPC, j % RPC]],
                             sem_s[b], add=True)
        return carry

    lax.fori_loop(1, EPC // 2, _hist, 0)
    for b in (0, 1):
        j = EPC - 2 + b
        pltpu.make_async_copy(ones_v, deg_sh.at[di_v.at[j // RPC, j % RPC]],
                              sem_s[b]).wait()
    plsc.subcore_barrier()

    pltpu.sync_copy(deg_sh.at[pl.ds(s * RPT, RPT)], stg_v)
    pltpu.async_copy(stg_v, out_hbm.at[c, pl.ds(s * RPT, RPT)], sem).wait()


def _deg_kernel(dst_r):
    return pl.kernel(
        _deg_body,
        out_type=jax.ShapeDtypeStruct((NC, NPAD, L), jnp.float32),
        mesh=_sc_mesh(),
        compiler_params=pltpu.CompilerParams(use_tc_tiling_on_sc=False),
        scratch_types=[
            pltpu.VMEM((2, RPC, CHUNK), jnp.int32),
            pltpu.VMEM((CHUNK, L), jnp.float32),
            pltpu.VMEM((RPT, L), jnp.float32),
            pltpu.VMEM_SHARED((NPAD, L), jnp.float32),
            pltpu.SemaphoreType.DMA,
            pltpu.SemaphoreType.DMA,
            pltpu.SemaphoreType.DMA,
        ],
    )(dst_r)


_PIECES = [(0, CHUNK), (CHUNK, CHUNK), (2 * CHUNK, CHUNK), (3 * CHUNK, CHUNK),
           (4 * CHUNK, RPT - 4 * CHUNK)]


def _agg_body(z_hbm, src_hbm, dst_hbm, out_hbm,
              si_v, di_v, r0, r1, r2, r3, zsp_sh, acc_sh,
              gi, gg0, gg1, gg2, gg3, gs0, gs1, gs2, gs3):
    c = lax.axis_index("c")
    s = lax.axis_index("s")
    rows = [r0, r1, r2, r3]
    sem_g = [gg0, gg1, gg2, gg3]
    sem_s = [gs0, gs1, gs2, gs3]

    pltpu.async_copy(src_hbm.at[s, 0], si_v, gi)
    pltpu.async_copy(dst_hbm.at[s, 0], di_v, gi)

    def _zero(i, carry):
        r1[i // (DH // L), pl.ds((i % (DH // L)) * L, L)] = jnp.zeros((L,), jnp.float32)
        return carry

    lax.fori_loop(0, CHUNK * (DH // L), _zero, 0, unroll=8)

    for off, ln in _PIECES:
        pltpu.async_copy(z_hbm.at[c, pl.ds(s * RPT + off, ln)],
                         r0.at[pl.ds(0, ln)], gg0).wait()
        pltpu.sync_copy(r0.at[pl.ds(0, ln)], zsp_sh.at[pl.ds(s * RPT + off, ln)])
        pltpu.sync_copy(r1.at[pl.ds(0, ln)], acc_sh.at[pl.ds(s * RPT + off, ln)])
    pltpu.make_async_copy(src_hbm.at[s, 0], si_v, gi).wait()
    pltpu.make_async_copy(dst_hbm.at[s, 0], di_v, gi).wait()
    plsc.subcore_barrier()

    def _steady(g, carry):
        for b in (0, 1, 2, 3):
            j = 4 * g + b
            pltpu.make_async_copy(rows[b], acc_sh.at[di_v.at[j - 4]],
                                  sem_s[b]).wait()
            pltpu.async_copy(zsp_sh.at[si_v.at[j]], rows[b], sem_g[b]).wait()
            pltpu.async_copy(rows[b], acc_sh.at[di_v.at[j]], sem_s[b], add=True)
        return carry

    for r in (0, 1, 2, 3):
        for b in (0, 1, 2, 3):
            pltpu.async_copy(zsp_sh.at[si_v.at[b]], rows[b], sem_g[b]).wait()
            pltpu.async_copy(rows[b], acc_sh.at[di_v.at[b]], sem_s[b], add=True)
        lax.fori_loop(1, RPC // 4, _steady, 0)
        for b in (0, 1, 2, 3):
            pltpu.make_async_copy(rows[b], acc_sh.at[di_v.at[RPC - 4 + b]],
                                  sem_s[b]).wait()
        if r < 3:
            pltpu.async_copy(src_hbm.at[s, r + 1], si_v, gi)
            pltpu.async_copy(dst_hbm.at[s, r + 1], di_v, gi)
            pltpu.make_async_copy(src_hbm.at[s, r + 1], si_v, gi).wait()
            pltpu.make_async_copy(dst_hbm.at[s, r + 1], di_v, gi).wait()
    plsc.subcore_barrier()

    for off, ln in _PIECES:
        pltpu.sync_copy(acc_sh.at[pl.ds(s * RPT + off, ln)], r0.at[pl.ds(0, ln)])
        pltpu.async_copy(r0.at[pl.ds(0, ln)],
                         out_hbm.at[c, pl.ds(s * RPT + off, ln)], gg0).wait()


def _agg_kernel(z_halves, src_r, dst_r):
    return pl.kernel(
        _agg_body,
        out_type=jax.ShapeDtypeStruct((2, NPAD, DH), jnp.float32),
        mesh=_sc_mesh(),
        compiler_params=pltpu.CompilerParams(use_tc_tiling_on_sc=False),
        scratch_types=(
            [pltpu.VMEM((RPC, CHUNK), jnp.int32)] * 2
            + [pltpu.VMEM((CHUNK, DH), jnp.float32)] * 4
            + [pltpu.VMEM_SHARED((NPAD, DH), jnp.float32)] * 2
            + [pltpu.SemaphoreType.DMA] * 9
        ),
    )(z_halves, src_r, dst_r)


def _prep_body(hist_ref, x0_ref, dis_ref, z_ref):
    cnt = hist_ref[0, :, 0:1] + hist_ref[1, :, 0:1]
    row = lax.broadcasted_iota(jnp.int32, (NPAD, 1), 0)
    dis = jnp.where(row < N, lax.rsqrt(cnt + 1.0), 0.0)
    dis_ref[...] = dis
    z0 = dis * x0_ref[...]
    z_ref[0] = z0[:, :DH]
    z_ref[1] = z0[:, DH:]


def _prep_kernel(hist_t, x0_p):
    return pl.pallas_call(
        _prep_body,
        out_shape=(
            jax.ShapeDtypeStruct((NPAD, 1), jnp.float32),
            jax.ShapeDtypeStruct((2, NPAD, DH), jnp.float32),
        ),
    )(hist_t, x0_p)


_GRID = 4
_RB = NPAD // _GRID


def _mid_body(u_ref, z_ref, dis_ref, w_ref, b_ref, x_ref, zn_ref):
    dis = dis_ref[...]
    y = dis * jnp.concatenate(
        [u_ref[0] + z_ref[0], u_ref[1] + z_ref[1]], axis=1)
    x = (jnp.dot(y, w_ref[...], preferred_element_type=jnp.float32)
         + b_ref[...][None, :])
    x_ref[...] = x
    zn = dis * x
    zn_ref[0] = zn[:, :DH]
    zn_ref[1] = zn[:, DH:]


def _mid_kernel(u, z, dis, w, b):
    return pl.pallas_call(
        _mid_body,
        grid=(_GRID,),
        in_specs=[
            pl.BlockSpec((2, _RB, DH), lambda i: (0, i, 0)),
            pl.BlockSpec((2, _RB, DH), lambda i: (0, i, 0)),
            pl.BlockSpec((_RB, 1), lambda i: (i, 0)),
            pl.BlockSpec((D, D), lambda i: (0, 0)),
            pl.BlockSpec((D,), lambda i: (0,)),
        ],
        out_specs=(
            pl.BlockSpec((_RB, D), lambda i: (i, 0)),
            pl.BlockSpec((2, _RB, DH), lambda i: (0, i, 0)),
        ),
        out_shape=(
            jax.ShapeDtypeStruct((NPAD, D), jnp.float32),
            jax.ShapeDtypeStruct((2, NPAD, DH), jnp.float32),
        ),
    )(u, z, dis, w, b)


def _fin_body(u_ref, z_ref, dis_ref, w_ref, b_ref, x0_ref, x1_ref,
              x2_ref, sum_ref):
    y = dis_ref[...] * jnp.concatenate(
        [u_ref[0] + z_ref[0], u_ref[1] + z_ref[1]], axis=1)
    x2 = (jnp.dot(y, w_ref[...], preferred_element_type=jnp.float32)
          + b_ref[...][None, :])
    x2_ref[...] = x2
    sum_ref[...] = x0_ref[...] + x1_ref[...] + x2


def _fin_kernel(u, z, dis, w, b, x0_p, x1_p):
    return pl.pallas_call(
        _fin_body,
        grid=(_GRID,),
        in_specs=[
            pl.BlockSpec((2, _RB, DH), lambda i: (0, i, 0)),
            pl.BlockSpec((2, _RB, DH), lambda i: (0, i, 0)),
            pl.BlockSpec((_RB, 1), lambda i: (i, 0)),
            pl.BlockSpec((D, D), lambda i: (0, 0)),
            pl.BlockSpec((D,), lambda i: (0,)),
            pl.BlockSpec((_RB, D), lambda i: (i, 0)),
            pl.BlockSpec((_RB, D), lambda i: (i, 0)),
        ],
        out_specs=(
            pl.BlockSpec((_RB, D), lambda i: (i, 0)),
            pl.BlockSpec((_RB, D), lambda i: (i, 0)),
        ),
        out_shape=(
            jax.ShapeDtypeStruct((NPAD, D), jnp.float32),
            jax.ShapeDtypeStruct((NPAD, D), jnp.float32),
        ),
    )(u, z, dis, w, b, x0_p, x1_p)


def kernel(item_emb, W1, b1, W2, b2, edge_index):
    ei = edge_index.astype(jnp.int32)
    pad = jnp.full((NW * EPT - E,), N, jnp.int32)
    src_r = jnp.concatenate([ei[0], pad]).reshape(NS, 4, RPC, CHUNK)
    dst_r = jnp.concatenate([ei[1], pad]).reshape(NS, 4, RPC, CHUNK)
    x0_p = jnp.pad(item_emb, ((0, NPAD - N), (0, 0)))

    hist = _deg_kernel(dst_r)
    dis, z0 = _prep_kernel(hist, x0_p)
    u1 = _agg_kernel(z0, src_r, dst_r)
    x1_p, z1 = _mid_kernel(u1, z0, dis, W1, b1)
    u2 = _agg_kernel(z1, src_r, dst_r)
    x2_p, summed_p = _fin_kernel(u2, z1, dis, W2, b2, x0_p, x1_p)

    return (summed_p[:N], item_emb, x1_p[:N], x2_p[:N])


# speedup vs baseline: 1.0058x; 1.0058x over previous
import functools

import jax
import jax.numpy as jnp
from jax import lax
from jax.experimental import pallas as pl
from jax.experimental.pallas import tpu as pltpu
from jax.experimental.pallas import tpu_sc as plsc

N = 10000
D = 128
E = 320000
NC = 2
NS = 16
NW = NC * NS
L = 16
DH = 64
CHUNK = 128
EPC = 80
EPC_IDX = EPC + 2
EPT = EPC * CHUNK
EPAD = NW * EPC_IDX * CHUNK
NPAD = 10112
RPT = NPAD // NS


def _sc_mesh():
    return plsc.VectorSubcoreMesh(core_axis_name="c", subcore_axis_name="s")


def _deg_body(dst_hbm, out_hbm, di_v, ones_v, stg_v, deg_sh, sem, ss0, ss1):
    c = lax.axis_index("c")
    s = lax.axis_index("s")
    sem_s = [ss0, ss1]
    pltpu.async_copy(dst_hbm.at[s, c], di_v, sem)

    e0 = jnp.where(lax.iota(jnp.int32, L) == 0, 1.0, 0.0).astype(jnp.float32)

    def _fill(i, carry):
        ones_v[i, :] = e0
        return carry

    lax.fori_loop(0, CHUNK, _fill, 0, unroll=8)

    def _zero(i, carry):
        stg_v[i, :] = jnp.zeros((L,), jnp.float32)
        return carry

    lax.fori_loop(0, RPT, _zero, 0, unroll=8)
    pltpu.sync_copy(stg_v, deg_sh.at[pl.ds(s * RPT, RPT)])
    pltpu.make_async_copy(dst_hbm.at[s, c], di_v, sem).wait()
    plsc.subcore_barrier()

    for b in (0, 1):
        pltpu.async_copy(ones_v, deg_sh.at[di_v.at[b]], sem_s[b], add=True)

    def _hist(g, carry):
        for b in (0, 1):
            j = 2 * g + b
            pltpu.make_async_copy(ones_v, deg_sh.at[di_v.at[j - 2]], sem_s[b]).wait()
            pltpu.async_copy(ones_v, deg_sh.at[di_v.at[j]], sem_s[b], add=True)
        return carry

    lax.fori_loop(1, EPC // 2, _hist, 0)
    for b in (0, 1):
        pltpu.make_async_copy(ones_v, deg_sh.at[di_v.at[EPC - 2 + b]], sem_s[b]).wait()
    plsc.subcore_barrier()

    pltpu.sync_copy(deg_sh.at[pl.ds(s * RPT, RPT)], stg_v)
    pltpu.async_copy(stg_v, out_hbm.at[c, pl.ds(s * RPT, RPT)], sem).wait()


def _deg_kernel(dst_r):
    return pl.kernel(
        _deg_body,
        out_type=jax.ShapeDtypeStruct((NC, NPAD, L), jnp.float32),
        mesh=_sc_mesh(),
        compiler_params=pltpu.CompilerParams(use_tc_tiling_on_sc=False),
        scratch_types=[
            pltpu.VMEM((EPC, CHUNK), jnp.int32),
            pltpu.VMEM((CHUNK, L), jnp.float32),
            pltpu.VMEM((RPT, L), jnp.float32),
            pltpu.VMEM_SHARED((NPAD, L), jnp.float32),
            pltpu.SemaphoreType.DMA,
            pltpu.SemaphoreType.DMA,
            pltpu.SemaphoreType.DMA,
        ],
    )(dst_r)


_PIECES = [(0, CHUNK), (CHUNK, CHUNK), (2 * CHUNK, CHUNK), (3 * CHUNK, CHUNK),
           (4 * CHUNK, RPT - 4 * CHUNK)]


def _agg_body(z_hbm, src_hbm, dst_hbm, out_hbm,
              si_v, di_v, r0, r1, zsp_sh, acc_sh,
              gi, gg0, gg1, gs0, gs1):
    c = lax.axis_index("c")
    s = lax.axis_index("s")
    rows = [r0, r1]
    sem_g = [gg0, gg1]
    sem_s = [gs0, gs1]

    pltpu.async_copy(src_hbm.at[s, 0], si_v, gi)
    pltpu.async_copy(dst_hbm.at[s, 0], di_v, gi)

    def _zero(i, carry):
        r1[i // (DH // L), pl.ds((i % (DH // L)) * L, L)] = jnp.zeros((L,), jnp.float32)
        return carry

    lax.fori_loop(0, CHUNK * (DH // L), _zero, 0, unroll=8)

    for off, ln in _PIECES:
        pltpu.async_copy(z_hbm.at[c, pl.ds(s * RPT + off, ln)],
                         r0.at[pl.ds(0, ln)], gg0).wait()
        pltpu.sync_copy(r0.at[pl.ds(0, ln)], zsp_sh.at[pl.ds(s * RPT + off, ln)])
        pltpu.sync_copy(r1.at[pl.ds(0, ln)], acc_sh.at[pl.ds(s * RPT + off, ln)])
    pltpu.make_async_copy(src_hbm.at[s, 0], si_v, gi).wait()
    pltpu.make_async_copy(dst_hbm.at[s, 0], di_v, gi).wait()
    plsc.subcore_barrier()

    def _steady(g, carry):
        for b in (0, 1):
            j = 2 * g + b
            pltpu.make_async_copy(rows[b], acc_sh.at[di_v.at[j - 2]],
                                  sem_s[b]).wait()
            pltpu.async_copy(zsp_sh.at[si_v.at[j]], rows[b], sem_g[b]).wait()
            pltpu.async_copy(rows[b], acc_sh.at[di_v.at[j]], sem_s[b], add=True)
        return carry

    for r in (0, 1):
        for b in (0, 1):
            pltpu.async_copy(zsp_sh.at[si_v.at[b]], rows[b], sem_g[b]).wait()
            pltpu.async_copy(rows[b], acc_sh.at[di_v.at[b]], sem_s[b], add=True)
        lax.fori_loop(1, EPC // 2, _steady, 0)
        for b in (0, 1):
            pltpu.make_async_copy(rows[b], acc_sh.at[di_v.at[EPC - 2 + b]],
                                  sem_s[b]).wait()
        if r == 0:
            pltpu.async_copy(src_hbm.at[s, 1], si_v, gi)
            pltpu.async_copy(dst_hbm.at[s, 1], di_v, gi)
            pltpu.make_async_copy(src_hbm.at[s, 1], si_v, gi).wait()
            pltpu.make_async_copy(dst_hbm.at[s, 1], di_v, gi).wait()
    plsc.subcore_barrier()

    for off, ln in _PIECES:
        pltpu.sync_copy(acc_sh.at[pl.ds(s * RPT + off, ln)], r0.at[pl.ds(0, ln)])
        pltpu.async_copy(r0.at[pl.ds(0, ln)],
                         out_hbm.at[c, pl.ds(s * RPT + off, ln)], gg0).wait()


def _agg_kernel(z_halves, src_r, dst_r):
    return pl.kernel(
        _agg_body,
        out_type=jax.ShapeDtypeStruct((2, NPAD, DH), jnp.float32),
        mesh=_sc_mesh(),
        compiler_params=pltpu.CompilerParams(use_tc_tiling_on_sc=False),
        scratch_types=(
            [pltpu.VMEM((EPC, CHUNK), jnp.int32)] * 2
            + [pltpu.VMEM((CHUNK, DH), jnp.float32)] * 2
            + [pltpu.VMEM_SHARED((NPAD, DH), jnp.float32)] * 2
            + [pltpu.SemaphoreType.DMA] * 5
        ),
    )(z_halves, src_r, dst_r)


def _prep_body(hist_ref, x0_ref, dis_ref, z_ref):
    cnt = hist_ref[0, :, 0:1] + hist_ref[1, :, 0:1]
    row = lax.broadcasted_iota(jnp.int32, (NPAD, 1), 0)
    dis = jnp.where(row < N, lax.rsqrt(cnt + 1.0), 0.0)
    dis_ref[...] = dis
    z0 = dis * x0_ref[...]
    z_ref[0] = z0[:, :DH]
    z_ref[1] = z0[:, DH:]


def _prep_kernel(hist_t, x0_p):
    return pl.pallas_call(
        _prep_body,
        out_shape=(
            jax.ShapeDtypeStruct((NPAD, 1), jnp.float32),
            jax.ShapeDtypeStruct((2, NPAD, DH), jnp.float32),
        ),
    )(hist_t, x0_p)


_GRID = 4
_RB = NPAD // _GRID


def _mid_body(u_ref, z_ref, dis_ref, w_ref, b_ref, x_ref, zn_ref):
    dis = dis_ref[...]
    y = dis * jnp.concatenate(
        [u_ref[0] + z_ref[0], u_ref[1] + z_ref[1]], axis=1)
    x = (jnp.dot(y, w_ref[...], preferred_element_type=jnp.float32)
         + b_ref[...][None, :])
    x_ref[...] = x
    zn = dis * x
    zn_ref[0] = zn[:, :DH]
    zn_ref[1] = zn[:, DH:]


def _mid_kernel(u, z, dis, w, b):
    return pl.pallas_call(
        _mid_body,
        grid=(_GRID,),
        in_specs=[
            pl.BlockSpec((2, _RB, DH), lambda i: (0, i, 0)),
            pl.BlockSpec((2, _RB, DH), lambda i: (0, i, 0)),
            pl.BlockSpec((_RB, 1), lambda i: (i, 0)),
            pl.BlockSpec((D, D), lambda i: (0, 0)),
            pl.BlockSpec((D,), lambda i: (0,)),
        ],
        out_specs=(
            pl.BlockSpec((_RB, D), lambda i: (i, 0)),
            pl.BlockSpec((2, _RB, DH), lambda i: (0, i, 0)),
        ),
        out_shape=(
            jax.ShapeDtypeStruct((NPAD, D), jnp.float32),
            jax.ShapeDtypeStruct((2, NPAD, DH), jnp.float32),
        ),
    )(u, z, dis, w, b)


def _fin_body(u_ref, z_ref, dis_ref, w_ref, b_ref, x0_ref, x1_ref,
              x2_ref, sum_ref):
    y = dis_ref[...] * jnp.concatenate(
        [u_ref[0] + z_ref[0], u_ref[1] + z_ref[1]], axis=1)
    x2 = (jnp.dot(y, w_ref[...], preferred_element_type=jnp.float32)
          + b_ref[...][None, :])
    x2_ref[...] = x2
    sum_ref[...] = x0_ref[...] + x1_ref[...] + x2


def _fin_kernel(u, z, dis, w, b, x0_p, x1_p):
    return pl.pallas_call(
        _fin_body,
        grid=(_GRID,),
        in_specs=[
            pl.BlockSpec((2, _RB, DH), lambda i: (0, i, 0)),
            pl.BlockSpec((2, _RB, DH), lambda i: (0, i, 0)),
            pl.BlockSpec((_RB, 1), lambda i: (i, 0)),
            pl.BlockSpec((D, D), lambda i: (0, 0)),
            pl.BlockSpec((D,), lambda i: (0,)),
            pl.BlockSpec((_RB, D), lambda i: (i, 0)),
            pl.BlockSpec((_RB, D), lambda i: (i, 0)),
        ],
        out_specs=(
            pl.BlockSpec((_RB, D), lambda i: (i, 0)),
            pl.BlockSpec((_RB, D), lambda i: (i, 0)),
        ),
        out_shape=(
            jax.ShapeDtypeStruct((NPAD, D), jnp.float32),
            jax.ShapeDtypeStruct((NPAD, D), jnp.float32),
        ),
    )(u, z, dis, w, b, x0_p, x1_p)


def kernel(item_emb, W1, b1, W2, b2, edge_index):
    ei = edge_index.astype(jnp.int32)
    pad = jnp.full((NW * EPT - E,), N, jnp.int32)
    src_r = jnp.concatenate([ei[0], pad]).reshape(NS, 2, EPC, CHUNK)
    dst_r = jnp.concatenate([ei[1], pad]).reshape(NS, 2, EPC, CHUNK)
    x0_p = jnp.pad(item_emb, ((0, NPAD - N), (0, 0)))

    hist = _deg_kernel(dst_r)
    dis, z0 = _prep_kernel(hist, x0_p)
    u1 = _agg_kernel(z0, src_r, dst_r)
    x1_p, z1 = _mid_kernel(u1, z0, dis, W1, b1)
    u2 = _agg_kernel(z1, src_r, dst_r)
    x2_p, summed_p = _fin_kernel(u2, z1, dis, W2, b2, x0_p, x1_p)

    return (summed_p[:N], item_emb, x1_p[:N], x2_p[:N])
